# baseline (speedup 1.0000x reference)
import jax
import jax.numpy as jnp
from jax.experimental import pallas as pl
from jax.experimental.pallas import tpu as pltpu


def _fused_body(q_ref, p_ref, t_ref, att_ref, sel_ref, idx_ref, *, k, n):
    q = q_ref[...]
    p = p_ref[...]
    t = t_ref[0]
    sim = jax.lax.dot_general(
        q, p, (((1,), (1,)), ((), ())),
        preferred_element_type=jnp.float32) / t

    h = n // 2
    a = sim[:, :h]
    b = sim[:, h:]
    pcols = jax.lax.broadcasted_iota(jnp.int32, (a.shape[0], h), 1)
    alo = a >= b
    vmax = jnp.where(alo, a, b)
    vmin = jnp.where(alo, b, a)
    cmax = jnp.where(alo, pcols, pcols + h)
    cmin = jnp.where(alo, pcols + h, pcols)
    vals, idxs = [], []
    for r in range(k):
        m = jnp.max(vmax, axis=1, keepdims=True)
        tc = jnp.min(jnp.where(vmax == m, cmax, n), axis=1,
                     keepdims=True)
        vals.append(m)
        idxs.append(tc)
        if r < k - 1:
            match = pcols == jnp.bitwise_and(tc, h - 1)
            vmax = jnp.where(match, vmin, vmax)
            cmax = jnp.where(match, cmin, cmax)
            vmin = jnp.where(match, -jnp.inf, vmin)
    v = jnp.concatenate(vals, axis=1)
    ix = jnp.concatenate(idxs, axis=1)
    e = jnp.exp(v - v[:, :1])
    inv_s = 1.0 / jnp.sum(e, axis=1, keepdims=True)
    att = jnp.where(sim >= v[:, k - 1:k],
                    jnp.exp(sim - v[:, :1]) * inv_s, 0.0)
    att_ref[...] = att
    sel_ref[...] = jax.lax.dot_general(
        att, p, (((1,), (0,)), ((), ())),
        preferred_element_type=jnp.float32)
    idx_ref[...] = ix


def kernel(query_embedding, prompts, temperature, top_k):
    del top_k
    b, d = query_embedding.shape
    n = prompts.shape[0]
    k = min(10, n)

    rt = 32
    attention, selected, idx = pl.pallas_call(
        lambda q, p, t, a, se, i: _fused_body(q, p, t, a, se, i, k=k, n=n),
        grid=(b // rt,),
        in_specs=[
            pl.BlockSpec((rt, d), lambda i: (i, 0)),
            pl.BlockSpec((n, d), lambda i: (0, 0)),
            pl.BlockSpec(memory_space=pltpu.SMEM),
        ],
        out_specs=[
            pl.BlockSpec((rt, n), lambda i: (i, 0)),
            pl.BlockSpec((rt, d), lambda i: (i, 0)),
            pl.BlockSpec((rt, k), lambda i: (i, 0)),
        ],
        out_shape=[
            jax.ShapeDtypeStruct((b, n), jnp.float32),
            jax.ShapeDtypeStruct((b, d), jnp.float32),
            jax.ShapeDtypeStruct((b, k), jnp.int32),
        ],
    )(query_embedding, prompts, temperature)

    return (selected, attention, idx)


# speedup vs baseline: 1.0913x; 1.0913x over previous
import jax
import jax.numpy as jnp
from jax.experimental import pallas as pl
from jax.experimental.pallas import tpu as pltpu

_NCHAINS = 4


def _fused_body(q_ref, pt_ref, t_ref, att_ref, sel_ref, idx_ref, *, k, n):
    q = q_ref[...]
    pt = pt_ref[...]
    t = t_ref[0]
    sim = jax.lax.dot_general(
        q, pt, (((1,), (0,)), ((), ())),
        preferred_element_type=jnp.float32) / t

    nc = _NCHAINS
    nq = n // nc
    colsq = jax.lax.broadcasted_iota(jnp.int32, (sim.shape[0], nq), 1)
    w = [sim[:, c * nq:(c + 1) * nq] for c in range(nc)]
    vq = [[] for _ in range(nc)]
    iq = [[] for _ in range(nc)]
    for r in range(k):
        for c in range(nc):
            m = jnp.max(w[c], axis=1, keepdims=True)
            il = jnp.min(jnp.where(w[c] == m, colsq, nq), axis=1,
                         keepdims=True)
            vq[c].append(m)
            iq[c].append(il + c * nq)
            if r < k - 1:
                w[c] = jnp.where(colsq == il, -jnp.inf, w[c])
    cv = jnp.concatenate([x for vs in vq for x in vs], axis=1)
    ci = jnp.concatenate([x for ixs in iq for x in ixs], axis=1)

    vals, idxs = [], []
    for r in range(k):
        m = jnp.max(cv, axis=1, keepdims=True)
        tc = jnp.min(jnp.where(cv == m, ci, n), axis=1, keepdims=True)
        vals.append(m)
        idxs.append(tc)
        if r < k - 1:
            cv = jnp.where(ci == tc, -jnp.inf, cv)
    v = jnp.concatenate(vals, axis=1)
    ix = jnp.concatenate(idxs, axis=1)

    e = jnp.exp(v - v[:, :1])
    inv_s = 1.0 / jnp.sum(e, axis=1, keepdims=True)
    att = jnp.where(sim >= v[:, k - 1:k],
                    jnp.exp(sim - v[:, :1]) * inv_s, 0.0)
    att_ref[...] = att
    sel_ref[...] = jax.lax.dot_general(
        att, pt, (((1,), (1,)), ((), ())),
        preferred_element_type=jnp.float32)
    idx_ref[...] = ix


def kernel(query_embedding, prompts, temperature, top_k):
    del top_k
    b, d = query_embedding.shape
    n = prompts.shape[0]
    k = min(10, n)
    pt = prompts.T

    rt = 64
    attention, selected, idx = pl.pallas_call(
        lambda qr, pr, tr, ar, sr, ir: _fused_body(
            qr, pr, tr, ar, sr, ir, k=k, n=n),
        grid=(b // rt,),
        in_specs=[
            pl.BlockSpec((rt, d), lambda i: (i, 0)),
            pl.BlockSpec((d, n), lambda i: (0, 0)),
            pl.BlockSpec(memory_space=pltpu.SMEM),
        ],
        out_specs=[
            pl.BlockSpec((rt, n), lambda i: (i, 0)),
            pl.BlockSpec((rt, d), lambda i: (i, 0)),
            pl.BlockSpec((rt, k), lambda i: (i, 0)),
        ],
        out_shape=[
            jax.ShapeDtypeStruct((b, n), jnp.float32),
            jax.ShapeDtypeStruct((b, d), jnp.float32),
            jax.ShapeDtypeStruct((b, k), jnp.int32),
        ],
        compiler_params=pltpu.CompilerParams(
            vmem_limit_bytes=100 * 1024 * 1024),
    )(query_embedding, pt, temperature)

    return (selected, attention, idx)


# speedup vs baseline: 1.1152x; 1.0219x over previous
import jax
import jax.numpy as jnp
from jax.experimental import pallas as pl
from jax.experimental.pallas import tpu as pltpu

_NCHAINS = 8


def _fused_body(q_ref, pt_ref, t_ref, att_ref, sel_ref, idx_ref, *, k, n):
    q = q_ref[...]
    pt = pt_ref[...]
    t = t_ref[0]
    sim = jax.lax.dot_general(
        q, pt, (((1,), (0,)), ((), ())),
        preferred_element_type=jnp.float32) / t

    nc = _NCHAINS
    nq = n // nc
    colsq = jax.lax.broadcasted_iota(jnp.int32, (sim.shape[0], nq), 1)
    w = [sim[:, c * nq:(c + 1) * nq] for c in range(nc)]
    vq = [[] for _ in range(nc)]
    iq = [[] for _ in range(nc)]
    for r in range(k):
        for c in range(nc):
            m = jnp.max(w[c], axis=1, keepdims=True)
            il = jnp.min(jnp.where(w[c] == m, colsq, nq), axis=1,
                         keepdims=True)
            vq[c].append(m)
            iq[c].append(il + c * nq)
            if r < k - 1:
                w[c] = jnp.where(colsq == il, -jnp.inf, w[c])
    cv = jnp.concatenate([x for vs in vq for x in vs], axis=1)
    ci = jnp.concatenate([x for ixs in iq for x in ixs], axis=1)

    vals, idxs = [], []
    for r in range(k):
        m = jnp.max(cv, axis=1, keepdims=True)
        tc = jnp.min(jnp.where(cv == m, ci, n), axis=1, keepdims=True)
        vals.append(m)
        idxs.append(tc)
        if r < k - 1:
            cv = jnp.where(ci == tc, -jnp.inf, cv)
    v = jnp.concatenate(vals, axis=1)
    ix = jnp.concatenate(idxs, axis=1)

    e = jnp.exp(v - v[:, :1])
    inv_s = 1.0 / jnp.sum(e, axis=1, keepdims=True)
    att = jnp.where(sim >= v[:, k - 1:k],
                    jnp.exp(sim - v[:, :1]) * inv_s, 0.0)
    att_ref[...] = att
    sel_ref[...] = jax.lax.dot_general(
        att, pt, (((1,), (1,)), ((), ())),
        preferred_element_type=jnp.float32)
    idx_ref[...] = ix


def kernel(query_embedding, prompts, temperature, top_k):
    del top_k
    b, d = query_embedding.shape
    n = prompts.shape[0]
    k = min(10, n)
    pt = prompts.T

    rt = 64
    attention, selected, idx = pl.pallas_call(
        lambda qr, pr, tr, ar, sr, ir: _fused_body(
            qr, pr, tr, ar, sr, ir, k=k, n=n),
        grid=(b // rt,),
        in_specs=[
            pl.BlockSpec((rt, d), lambda i: (i, 0)),
            pl.BlockSpec((d, n), lambda i: (0, 0)),
            pl.BlockSpec(memory_space=pltpu.SMEM),
        ],
        out_specs=[
            pl.BlockSpec((rt, n), lambda i: (i, 0)),
            pl.BlockSpec((rt, d), lambda i: (i, 0)),
            pl.BlockSpec((rt, k), lambda i: (i, 0)),
        ],
        out_shape=[
            jax.ShapeDtypeStruct((b, n), jnp.float32),
            jax.ShapeDtypeStruct((b, d), jnp.float32),
            jax.ShapeDtypeStruct((b, k), jnp.int32),
        ],
        compiler_params=pltpu.CompilerParams(
            vmem_limit_bytes=100 * 1024 * 1024),
    )(query_embedding, pt, temperature)

    return (selected, attention, idx)


# speedup vs baseline: 1.2670x; 1.1361x over previous
import jax
import jax.numpy as jnp
from jax.experimental import pallas as pl
from jax.experimental.pallas import tpu as pltpu

_NCHAINS = 8


def _fused_body(q_ref, pt_ref, t_ref, att_ref, sel_ref, idx_ref, *, k, n):
    q = q_ref[...]
    pt = pt_ref[...]
    t = t_ref[0]
    sim = jax.lax.dot_general(
        q, pt, (((1,), (0,)), ((), ())),
        preferred_element_type=jnp.float32) / t

    nc = _NCHAINS
    nq = n // nc
    hq = nq // 2
    colsh = jax.lax.broadcasted_iota(jnp.int32, (sim.shape[0], hq), 1)
    vmax, vmin, cmax, cmin = [], [], [], []
    for c in range(nc):
        a = sim[:, c * nq:c * nq + hq]
        bb = sim[:, c * nq + hq:(c + 1) * nq]
        alo = a >= bb
        vmax.append(jnp.where(alo, a, bb))
        vmin.append(jnp.where(alo, bb, a))
        cmax.append(jnp.where(alo, colsh, colsh + hq))
        cmin.append(jnp.where(alo, colsh + hq, colsh))
    vq = [[] for _ in range(nc)]
    iq = [[] for _ in range(nc)]
    for r in range(k):
        for c in range(nc):
            m = jnp.max(vmax[c], axis=1, keepdims=True)
            il = jnp.min(jnp.where(vmax[c] == m, cmax[c], nq), axis=1,
                         keepdims=True)
            vq[c].append(m)
            iq[c].append(il + c * nq)
            if r < k - 1:
                match = colsh == jnp.bitwise_and(il, hq - 1)
                vmax[c] = jnp.where(match, vmin[c], vmax[c])
                cmax[c] = jnp.where(match, cmin[c], cmax[c])
                vmin[c] = jnp.where(match, -jnp.inf, vmin[c])
    cv = jnp.concatenate([x for vs in vq for x in vs], axis=1)
    ci = jnp.concatenate([x for ixs in iq for x in ixs], axis=1)

    vals, idxs = [], []
    for r in range(k):
        m = jnp.max(cv, axis=1, keepdims=True)
        tc = jnp.min(jnp.where(cv == m, ci, n), axis=1, keepdims=True)
        vals.append(m)
        idxs.append(tc)
        if r < k - 1:
            cv = jnp.where(ci == tc, -jnp.inf, cv)
    v = jnp.concatenate(vals, axis=1)
    ix = jnp.concatenate(idxs, axis=1)

    e = jnp.exp(v - v[:, :1])
    inv_s = 1.0 / jnp.sum(e, axis=1, keepdims=True)
    att = jnp.where(sim >= v[:, k - 1:k],
                    jnp.exp(sim - v[:, :1]) * inv_s, 0.0)
    att_ref[...] = att
    sel_ref[...] = jax.lax.dot_general(
        att, pt, (((1,), (1,)), ((), ())),
        preferred_element_type=jnp.float32)
    idx_ref[...] = ix


def kernel(query_embedding, prompts, temperature, top_k):
    del top_k
    b, d = query_embedding.shape
    n = prompts.shape[0]
    k = min(10, n)
    pt = prompts.T

    rt = 64
    attention, selected, idx = pl.pallas_call(
        lambda qr, pr, tr, ar, sr, ir: _fused_body(
            qr, pr, tr, ar, sr, ir, k=k, n=n),
        grid=(b // rt,),
        in_specs=[
            pl.BlockSpec((rt, d), lambda i: (i, 0)),
            pl.BlockSpec((d, n), lambda i: (0, 0)),
            pl.BlockSpec(memory_space=pltpu.SMEM),
        ],
        out_specs=[
            pl.BlockSpec((rt, n), lambda i: (i, 0)),
            pl.BlockSpec((rt, d), lambda i: (i, 0)),
            pl.BlockSpec((rt, k), lambda i: (i, 0)),
        ],
        out_shape=[
            jax.ShapeDtypeStruct((b, n), jnp.float32),
            jax.ShapeDtypeStruct((b, d), jnp.float32),
            jax.ShapeDtypeStruct((b, k), jnp.int32),
        ],
        compiler_params=pltpu.CompilerParams(
            vmem_limit_bytes=100 * 1024 * 1024),
    )(query_embedding, pt, temperature)

    return (selected, attention, idx)


# speedup vs baseline: 1.4499x; 1.1443x over previous
import jax
import jax.numpy as jnp
from jax.experimental import pallas as pl
from jax.experimental.pallas import tpu as pltpu

_NCHAINS = 8


def _fused_body(q_ref, pt_ref, t_ref, att_ref, sel_ref, idx_ref, *, k, n):
    q = q_ref[...]
    pt = pt_ref[...]
    t = t_ref[0]
    sim = jax.lax.dot_general(
        q, pt, (((1,), (0,)), ((), ())),
        preferred_element_type=jnp.float32) / t

    nc = _NCHAINS
    nq = n // nc
    hq = nq // 2
    colsh = jax.lax.broadcasted_iota(
        jnp.int32, (sim.shape[0], hq), 1).astype(jnp.float32)
    vmax, vmin, cmax, cmin = [], [], [], []
    for c in range(nc):
        a = sim[:, c * nq:c * nq + hq]
        bb = sim[:, c * nq + hq:(c + 1) * nq]
        alo = a >= bb
        vmax.append(jnp.where(alo, a, bb))
        vmin.append(jnp.where(alo, bb, a))
        cmax.append(jnp.where(alo, colsh, colsh + hq))
        cmin.append(jnp.where(alo, colsh + hq, colsh))
    vq = [[] for _ in range(nc)]
    iq = [[] for _ in range(nc)]
    for r in range(k):
        for c in range(nc):
            m = jnp.max(vmax[c], axis=1, keepdims=True)
            il = jnp.min(jnp.where(vmax[c] == m, cmax[c], float(nq)),
                         axis=1, keepdims=True)
            vq[c].append(m)
            iq[c].append(il + c * nq)
            if r < k - 1:
                match = colsh == jnp.where(il >= hq, il - hq, il)
                vmax[c] = jnp.where(match, vmin[c], vmax[c])
                cmax[c] = jnp.where(match, cmin[c], cmax[c])
                vmin[c] = jnp.where(match, -jnp.inf, vmin[c])
    cv = jnp.concatenate([x for vs in vq for x in vs], axis=1)
    ci = jnp.concatenate([x for ixs in iq for x in ixs], axis=1)

    vals, idxs = [], []
    for r in range(k):
        m = jnp.max(cv, axis=1, keepdims=True)
        tc = jnp.min(jnp.where(cv == m, ci, float(n)), axis=1, keepdims=True)
        vals.append(m)
        idxs.append(tc)
        if r < k - 1:
            cv = jnp.where(ci == tc, -jnp.inf, cv)
    v = jnp.concatenate(vals, axis=1)
    ix = jnp.concatenate(idxs, axis=1).astype(jnp.int32)

    e = jnp.exp(v - v[:, :1])
    inv_s = 1.0 / jnp.sum(e, axis=1, keepdims=True)
    att = jnp.where(sim >= v[:, k - 1:k],
                    jnp.exp(sim - v[:, :1]) * inv_s, 0.0)
    att_ref[...] = att
    sel_ref[...] = jax.lax.dot_general(
        att, pt, (((1,), (1,)), ((), ())),
        preferred_element_type=jnp.float32)
    idx_ref[...] = ix


def kernel(query_embedding, prompts, temperature, top_k):
    del top_k
    b, d = query_embedding.shape
    n = prompts.shape[0]
    k = min(10, n)
    pt = prompts.T

    rt = 64
    attention, selected, idx = pl.pallas_call(
        lambda qr, pr, tr, ar, sr, ir: _fused_body(
            qr, pr, tr, ar, sr, ir, k=k, n=n),
        grid=(b // rt,),
        in_specs=[
            pl.BlockSpec((rt, d), lambda i: (i, 0)),
            pl.BlockSpec((d, n), lambda i: (0, 0)),
            pl.BlockSpec(memory_space=pltpu.SMEM),
        ],
        out_specs=[
            pl.BlockSpec((rt, n), lambda i: (i, 0)),
            pl.BlockSpec((rt, d), lambda i: (i, 0)),
            pl.BlockSpec((rt, k), lambda i: (i, 0)),
        ],
        out_shape=[
            jax.ShapeDtypeStruct((b, n), jnp.float32),
            jax.ShapeDtypeStruct((b, d), jnp.float32),
            jax.ShapeDtypeStruct((b, k), jnp.int32),
        ],
        compiler_params=pltpu.CompilerParams(
            vmem_limit_bytes=100 * 1024 * 1024),
    )(query_embedding, pt, temperature)

    return (selected, attention, idx)
